# Initial kernel scaffold; baseline (speedup 1.0000x reference)
import jax
import jax.numpy as jnp
from jax.experimental import pallas as pl


def kernel(x_input, edge_index_input, pos_edge_index_input, Wl, bl, Wg, bg, Wb, bb):
    raise NotImplementedError("write your pallas kernel here")


# speedup vs baseline: 8.2701x; 8.2701x over previous
import functools

import jax
import jax.numpy as jnp
from jax import lax
from jax.experimental import pallas as pl
from jax.experimental.pallas import tpu as pltpu
from jax.experimental.pallas import tpu_sc as plsc

N = 10000
E = 320000
D = 128

NC = 2
NS = 16
NW = NC * NS
EW = E // NW
K = 80
NCHUNK = EW // K

_f32 = jnp.float32


def _k1_body(x_ref, wl_ref, bl_ref, wg_ref, h2_ref):
    h = jnp.dot(x_ref[...], wl_ref[...], preferred_element_type=_f32)
    h = h + bl_ref[...]
    h2_ref[...] = jnp.dot(h, wg_ref[...], preferred_element_type=_f32)


def _tc_h2(x, Wl, bl2, Wg):
    blk = 1000
    return pl.pallas_call(
        _k1_body,
        grid=(N // blk,),
        in_specs=[
            pl.BlockSpec((blk, D), lambda i: (i, 0)),
            pl.BlockSpec((D, D), lambda i: (0, 0)),
            pl.BlockSpec((1, D), lambda i: (0, 0)),
            pl.BlockSpec((D, D), lambda i: (0, 0)),
        ],
        out_specs=pl.BlockSpec((blk, D), lambda i: (i, 0)),
        out_shape=jax.ShapeDtypeStruct((N, D), _f32),
    )(x, Wl, bl2, Wg)


def _k2_body(h2_ref, d0_ref, d1_ref, g_ref):
    deg = d0_ref[...] + d1_ref[...] + 1.0
    dinv = lax.rsqrt(deg)
    g_ref[...] = h2_ref[...] * dinv


def _tc_prescale(h2, d0, d1):
    blk = 1000
    return pl.pallas_call(
        _k2_body,
        grid=(N // blk,),
        in_specs=[
            pl.BlockSpec((blk, D), lambda i: (i, 0)),
            pl.BlockSpec((blk, 1), lambda i: (i, 0)),
            pl.BlockSpec((blk, 1), lambda i: (i, 0)),
        ],
        out_specs=pl.BlockSpec((blk, D), lambda i: (i, 0)),
        out_shape=jax.ShapeDtypeStruct((N, D), _f32),
    )(h2, d0, d1)


def _k4_body(a0_ref, a1_ref, g_ref, d0_ref, d1_ref, bg_ref, wb_ref,
             hf_ref, t_ref):
    deg = d0_ref[...] + d1_ref[...] + 1.0
    dinv = lax.rsqrt(deg)
    hf = dinv * (a0_ref[...] + a1_ref[...] + g_ref[...]) + bg_ref[...]
    hf_ref[...] = hf
    t_ref[...] = lax.dot_general(hf, wb_ref[...],
                                 (((1,), (1,)), ((), ())),
                                 preferred_element_type=_f32)


def _tc_finalize(a0, a1, g, d0, d1, bg2, Wb0):
    blk = 1000
    return pl.pallas_call(
        _k4_body,
        grid=(N // blk,),
        in_specs=[
            pl.BlockSpec((blk, D), lambda i: (i, 0)),
            pl.BlockSpec((blk, D), lambda i: (i, 0)),
            pl.BlockSpec((blk, D), lambda i: (i, 0)),
            pl.BlockSpec((blk, 1), lambda i: (i, 0)),
            pl.BlockSpec((blk, 1), lambda i: (i, 0)),
            pl.BlockSpec((1, D), lambda i: (0, 0)),
            pl.BlockSpec((D, D), lambda i: (0, 0)),
        ],
        out_specs=[
            pl.BlockSpec((blk, D), lambda i: (i, 0)),
            pl.BlockSpec((blk, D), lambda i: (i, 0)),
        ],
        out_shape=[
            jax.ShapeDtypeStruct((N, D), _f32),
            jax.ShapeDtypeStruct((N, D), _f32),
        ],
    )(a0, a1, g, d0, d1, bg2, Wb0)


_SC_MESH = plsc.VectorSubcoreMesh(core_axis_name="c", subcore_axis_name="s")

NPAD = 10240


def _deg_body(dst_hbm, degout_hbm, idx_v, ones_v, zb_v, deg_sh, sem):
    cid = lax.axis_index("c")
    sid = lax.axis_index("s")
    w = sid * NC + cid
    base = w * EW

    zeros16 = jnp.zeros((16,), _f32)
    ones16 = jnp.ones((16,), _f32)
    for j in range(1024 // 16):
        zb_v[pl.ds(j * 16, 16)] = zeros16
    for j in range(K // 16):
        ones_v[pl.ds(j * 16, 16)] = ones16

    @pl.when(sid < 10)
    def _():
        pltpu.sync_copy(zb_v, deg_sh.at[pl.ds(sid * 1024, 1024)])

    plsc.subcore_barrier()

    def chunk(c, _):
        pltpu.sync_copy(dst_hbm.at[pl.ds(base + c * K, K)], idx_v)
        pltpu.sync_copy(ones_v, deg_sh.at[idx_v], add=True)
        return _

    lax.fori_loop(0, NCHUNK, chunk, None)
    plsc.subcore_barrier()

    @pl.when(sid < 10)
    def _():
        pltpu.sync_copy(deg_sh.at[pl.ds(sid * 1024, 1024)],
                        degout_hbm.at[pl.ds(cid * NPAD + sid * 1024, 1024)])


@functools.partial(
    pl.kernel,
    out_type=jax.ShapeDtypeStruct((NC * NPAD,), _f32),
    mesh=_SC_MESH,
    scratch_types=[
        pltpu.VMEM((K,), jnp.int32),
        pltpu.VMEM((K,), _f32),
        pltpu.VMEM((1024,), _f32),
        pltpu.VMEM_SHARED((NPAD,), _f32),
        pltpu.SemaphoreType.DMA,
    ],
)
def _sc_degree(dst_hbm, degout_hbm, idx_v, ones_v, zb_v, deg_sh, sem):
    _deg_body(dst_hbm, degout_hbm, idx_v, ones_v, zb_v, deg_sh, sem)


ZROWS = 200


def _msg_body(src_hbm, dst_hbm, g_hbm, aggout_hbm,
              idxs_v, idxd_v, rows_v, zb_v, agg_sh, sem):
    cid = lax.axis_index("c")
    sid = lax.axis_index("s")
    w = sid * NC + cid
    base = w * EW

    zeros16 = jnp.zeros((16,), _f32)

    def zrow(r, _):
        for j in range(D // 16):
            zb_v[r, pl.ds(j * 16, 16)] = zeros16
        return _

    lax.fori_loop(0, ZROWS, zrow, None)

    @pl.when(sid < 10)
    def _():
        for r4 in range(5):
            pltpu.sync_copy(
                zb_v, agg_sh.at[pl.ds(sid * 1000 + r4 * ZROWS, ZROWS)])
    plsc.subcore_barrier()

    def chunk(c, _):
        pltpu.sync_copy(src_hbm.at[pl.ds(base + c * K, K)], idxs_v)
        pltpu.sync_copy(dst_hbm.at[pl.ds(base + c * K, K)], idxd_v)
        pltpu.async_copy(g_hbm.at[idxs_v], rows_v, sem).wait()
        pltpu.sync_copy(rows_v, agg_sh.at[idxd_v], add=True)
        return _

    lax.fori_loop(0, NCHUNK, chunk, None)
    plsc.subcore_barrier()

    @pl.when(sid < 10)
    def _():
        for r4 in range(5):
            r0 = sid * 1000 + r4 * ZROWS
            pltpu.sync_copy(agg_sh.at[pl.ds(r0, ZROWS)],
                            aggout_hbm.at[cid, pl.ds(r0, ZROWS)])


@functools.partial(
    pl.kernel,
    out_type=jax.ShapeDtypeStruct((NC, N, D), _f32),
    mesh=_SC_MESH,
    scratch_types=[
        pltpu.VMEM((K,), jnp.int32),
        pltpu.VMEM((K,), jnp.int32),
        pltpu.VMEM((K, D), _f32),
        pltpu.VMEM((ZROWS, D), _f32),
        pltpu.VMEM_SHARED((N, D), _f32),
        pltpu.SemaphoreType.DMA,
    ],
)
def _sc_message(src_hbm, dst_hbm, g_hbm, aggout_hbm,
                idxs_v, idxd_v, rows_v, zb_v, agg_sh, sem):
    _msg_body(src_hbm, dst_hbm, g_hbm, aggout_hbm,
              idxs_v, idxd_v, rows_v, zb_v, agg_sh, sem)


def _bil_body(i0_hbm, i1_hbm, hf_hbm, t_hbm, bb_hbm, out_hbm,
              idx0_v, idx1_v, r1_v, r2_v, sv_v, bb_v, sem1, sem2):
    cid = lax.axis_index("c")
    sid = lax.axis_index("s")
    w = sid * NC + cid
    base = w * EW

    pltpu.sync_copy(bb_hbm, bb_v)
    bbvec = bb_v[...]
    lane = lax.iota(jnp.int32, 16)
    perms = [jnp.bitwise_xor(lane, 1 << p) for p in range(4)]

    def _allsum(v):
        for p in perms:
            v = v + v.at[p].get(mode="promise_in_bounds")
        return v

    def chunk(c, _):
        pltpu.sync_copy(i0_hbm.at[pl.ds(base + c * K, K)], idx0_v)
        pltpu.sync_copy(i1_hbm.at[pl.ds(base + c * K, K)], idx1_v)
        pltpu.async_copy(hf_hbm.at[idx0_v], r1_v, sem1).wait()
        pltpu.async_copy(t_hbm.at[idx1_v], r2_v, sem2).wait()

        def group(gidx, _):
            acc = jnp.zeros((16,), _f32)
            for e in range(16):
                edge = gidx * 16 + e
                v = r1_v[edge, pl.ds(0, 16)] * r2_v[edge, pl.ds(0, 16)]
                for j in range(1, D // 16):
                    v = v + (r1_v[edge, pl.ds(j * 16, 16)]
                             * r2_v[edge, pl.ds(j * 16, 16)])
                s = _allsum(v)
                acc = jnp.where(lane == e, s, acc)
            sv_v[pl.ds(gidx * 16, 16)] = acc + bbvec
            return _

        lax.fori_loop(0, K // 16, group, None)
        pltpu.sync_copy(sv_v, out_hbm.at[pl.ds(base + c * K, K)])
        return _

    lax.fori_loop(0, NCHUNK, chunk, None)


@functools.partial(
    pl.kernel,
    out_type=jax.ShapeDtypeStruct((E,), _f32),
    mesh=_SC_MESH,
    scratch_types=[
        pltpu.VMEM((K,), jnp.int32),
        pltpu.VMEM((K,), jnp.int32),
        pltpu.VMEM((K, D), _f32),
        pltpu.VMEM((K, D), _f32),
        pltpu.VMEM((K,), _f32),
        pltpu.VMEM((16,), _f32),
        pltpu.SemaphoreType.DMA,
        pltpu.SemaphoreType.DMA,
    ],
)
def _sc_bilinear(i0_hbm, i1_hbm, hf_hbm, t_hbm, bb_hbm, out_hbm,
                 idx0_v, idx1_v, r1_v, r2_v, sv_v, bb_v, sem1, sem2):
    _bil_body(i0_hbm, i1_hbm, hf_hbm, t_hbm, bb_hbm, out_hbm,
              idx0_v, idx1_v, r1_v, r2_v, sv_v, bb_v, sem1, sem2)


def kernel(x_input, edge_index_input, pos_edge_index_input,
           Wl, bl, Wg, bg, Wb, bb):
    src = pos_edge_index_input[0]
    dst = pos_edge_index_input[1]
    i0 = edge_index_input[0]
    i1 = edge_index_input[1]
    bl2 = bl.reshape(1, D)
    bg2 = bg.reshape(1, D)
    Wb0 = Wb[0]
    bb16 = jnp.broadcast_to(bb.astype(_f32), (16,))

    h2 = _tc_h2(x_input, Wl, bl2, Wg)
    degp = _sc_degree(dst)
    d0 = degp[0:N].reshape(N, 1)
    d1 = degp[NPAD:NPAD + N].reshape(N, 1)
    g = _tc_prescale(h2, d0, d1)
    aggp = _sc_message(src, dst, g)
    hf, t = _tc_finalize(aggp[0], aggp[1], g, d0, d1, bg2, Wb0)
    scores = _sc_bilinear(i0, i1, hf, t, bb16)
    return scores


# speedup vs baseline: 14.5561x; 1.7601x over previous
import functools

import jax
import jax.numpy as jnp
from jax import lax
from jax.experimental import pallas as pl
from jax.experimental.pallas import tpu as pltpu
from jax.experimental.pallas import tpu_sc as plsc

N = 10000
E = 320000
D = 128

NC = 2
NS = 16
NW = NC * NS
EW = E // NW
K = 80
NCHUNK = EW // K

_f32 = jnp.float32


def _k1_body(x_ref, wl_ref, bl_ref, wg_ref, h2_ref):
    h = jnp.dot(x_ref[...], wl_ref[...], preferred_element_type=_f32)
    h = h + bl_ref[...]
    h2_ref[...] = jnp.dot(h, wg_ref[...], preferred_element_type=_f32)


def _tc_h2(x, Wl, bl2, Wg):
    blk = 1000
    return pl.pallas_call(
        _k1_body,
        grid=(N // blk,),
        in_specs=[
            pl.BlockSpec((blk, D), lambda i: (i, 0)),
            pl.BlockSpec((D, D), lambda i: (0, 0)),
            pl.BlockSpec((1, D), lambda i: (0, 0)),
            pl.BlockSpec((D, D), lambda i: (0, 0)),
        ],
        out_specs=pl.BlockSpec((blk, D), lambda i: (i, 0)),
        out_shape=jax.ShapeDtypeStruct((N, D), _f32),
    )(x, Wl, bl2, Wg)


def _k2_body(h2_ref, d0_ref, d1_ref, g_ref):
    deg = d0_ref[...] + d1_ref[...] + 1.0
    dinv = lax.rsqrt(deg)
    g_ref[...] = h2_ref[...] * dinv


def _tc_prescale(h2, d0, d1):
    blk = 1000
    return pl.pallas_call(
        _k2_body,
        grid=(N // blk,),
        in_specs=[
            pl.BlockSpec((blk, D), lambda i: (i, 0)),
            pl.BlockSpec((blk, 1), lambda i: (i, 0)),
            pl.BlockSpec((blk, 1), lambda i: (i, 0)),
        ],
        out_specs=pl.BlockSpec((blk, D), lambda i: (i, 0)),
        out_shape=jax.ShapeDtypeStruct((N, D), _f32),
    )(h2, d0, d1)


def _k4_body(a0_ref, a1_ref, g_ref, d0_ref, d1_ref, bg_ref, wb_ref,
             hf_ref, t_ref):
    deg = d0_ref[...] + d1_ref[...] + 1.0
    dinv = lax.rsqrt(deg)
    hf = dinv * (a0_ref[...] + a1_ref[...] + g_ref[...]) + bg_ref[...]
    hf_ref[...] = hf
    t_ref[...] = lax.dot_general(hf, wb_ref[...],
                                 (((1,), (1,)), ((), ())),
                                 preferred_element_type=_f32)


def _tc_finalize(a0, a1, g, d0, d1, bg2, Wb0):
    blk = 1000
    return pl.pallas_call(
        _k4_body,
        grid=(N // blk,),
        in_specs=[
            pl.BlockSpec((blk, D), lambda i: (i, 0)),
            pl.BlockSpec((blk, D), lambda i: (i, 0)),
            pl.BlockSpec((blk, D), lambda i: (i, 0)),
            pl.BlockSpec((blk, 1), lambda i: (i, 0)),
            pl.BlockSpec((blk, 1), lambda i: (i, 0)),
            pl.BlockSpec((1, D), lambda i: (0, 0)),
            pl.BlockSpec((D, D), lambda i: (0, 0)),
        ],
        out_specs=[
            pl.BlockSpec((blk, D), lambda i: (i, 0)),
            pl.BlockSpec((blk, D), lambda i: (i, 0)),
        ],
        out_shape=[
            jax.ShapeDtypeStruct((N, D), _f32),
            jax.ShapeDtypeStruct((N, D), _f32),
        ],
    )(a0, a1, g, d0, d1, bg2, Wb0)


_SC_MESH = plsc.VectorSubcoreMesh(core_axis_name="c", subcore_axis_name="s")

NPAD = 10240


def _deg_body(dst_hbm, degout_hbm, idx_v, ones_v, zb_v, deg_sh,
              sA0, sA1, sC0, sC1):
    cid = lax.axis_index("c")
    sid = lax.axis_index("s")
    w = sid * NC + cid
    base = w * EW
    sA = [sA0, sA1]
    sC = [sC0, sC1]

    zeros16 = jnp.zeros((16,), _f32)
    ones16 = jnp.ones((16,), _f32)
    for j in range(1024 // 16):
        zb_v[pl.ds(j * 16, 16)] = zeros16
    for j in range(K // 16):
        ones_v[pl.ds(j * 16, 16)] = ones16

    @pl.when(sid < 10)
    def _():
        pltpu.sync_copy(zb_v, deg_sh.at[pl.ds(sid * 1024, 1024)])

    plsc.subcore_barrier()

    def cpA(c, s):
        return pltpu.make_async_copy(
            dst_hbm.at[pl.ds(base + c * K, K)], idx_v.at[s], sA[s])

    def cpC(c, s):
        return pltpu.make_async_copy(ones_v, deg_sh.at[idx_v.at[s]], sC[s])

    cpA(0, 0).start()
    cpA(1, 1).start()

    def pair(i, _):
        c = i * 2
        for s in (0, 1):
            cc = c + s
            cpA(cc, s).wait()
            cpC(cc, s).start(add=True)
            @pl.when(cc + 2 < NCHUNK)
            def _():
                cpC(cc, s).wait()
                cpA(cc + 2, s).start()
        return _

    lax.fori_loop(0, NCHUNK // 2, pair, None)
    cpA(NCHUNK - 1, 0).wait()
    cpC(NCHUNK - 1, 0).start(add=True)
    cpC(NCHUNK - 1, 0).wait()
    cpC(NCHUNK - 2, 1).wait()

    plsc.subcore_barrier()

    @pl.when(sid < 10)
    def _():
        pltpu.sync_copy(deg_sh.at[pl.ds(sid * 1024, 1024)],
                        degout_hbm.at[pl.ds(cid * NPAD + sid * 1024, 1024)])


@functools.partial(
    pl.kernel,
    out_type=jax.ShapeDtypeStruct((NC * NPAD,), _f32),
    mesh=_SC_MESH,
    scratch_types=[
        pltpu.VMEM((2, K), jnp.int32),
        pltpu.VMEM((K,), _f32),
        pltpu.VMEM((1024,), _f32),
        pltpu.VMEM_SHARED((NPAD,), _f32),
        pltpu.SemaphoreType.DMA,
        pltpu.SemaphoreType.DMA,
        pltpu.SemaphoreType.DMA,
        pltpu.SemaphoreType.DMA,
    ],
)
def _sc_degree(dst_hbm, degout_hbm, idx_v, ones_v, zb_v, deg_sh,
               sA0, sA1, sC0, sC1):
    _deg_body(dst_hbm, degout_hbm, idx_v, ones_v, zb_v, deg_sh,
              sA0, sA1, sC0, sC1)


ZROWS = 200


def _msg_body(src_hbm, dst_hbm, g_hbm, aggout_hbm,
              idxs_v, idxd_v, rows_v, zb_v, agg_sh,
              sAs0, sAs1, sAd0, sAd1, sB0, sB1):
    cid = lax.axis_index("c")
    sid = lax.axis_index("s")
    w = sid * NC + cid
    base = w * EW
    sAs = [sAs0, sAs1]
    sAd = [sAd0, sAd1]
    sB = [sB0, sB1]

    zeros16 = jnp.zeros((16,), _f32)

    def zrow(r, _):
        for j in range(D // 16):
            zb_v[r, pl.ds(j * 16, 16)] = zeros16
        return _

    lax.fori_loop(0, ZROWS, zrow, None)

    @pl.when(sid < 10)
    def _():
        for r4 in range(5):
            pltpu.sync_copy(
                zb_v, agg_sh.at[pl.ds(sid * 1000 + r4 * ZROWS, ZROWS)])
    plsc.subcore_barrier()

    def cpAs(c, s):
        return pltpu.make_async_copy(
            src_hbm.at[pl.ds(base + c * K, K)], idxs_v.at[s], sAs[s])

    def cpAd(c, s):
        return pltpu.make_async_copy(
            dst_hbm.at[pl.ds(base + c * K, K)], idxd_v.at[s], sAd[s])

    def cpB(c, s):
        return pltpu.make_async_copy(
            g_hbm.at[idxs_v.at[s]], rows_v.at[s], sB[s])

    cpAs(0, 0).start(); cpAd(0, 0).start()
    cpAs(1, 1).start(); cpAd(1, 1).start()
    cpAs(0, 0).wait(); cpB(0, 0).start()

    def pair(i, _):
        c = i * 2
        for s in (0, 1):
            cc = c + s
            ns = 1 - s
            cpB(cc, s).wait()
            @pl.when(cc + 1 < NCHUNK)
            def _():
                cpAs(cc + 1, ns).wait()
                cpB(cc + 1, ns).start()
            cpAd(cc, s).wait()
            pltpu.sync_copy(rows_v.at[s], agg_sh.at[idxd_v.at[s]], add=True)
            @pl.when(cc + 2 < NCHUNK)
            def _():
                cpAs(cc + 2, s).start()
                cpAd(cc + 2, s).start()
        return _

    lax.fori_loop(0, NCHUNK // 2, pair, None)
    cc = NCHUNK - 1
    cpB(cc, 0).wait()
    cpAd(cc, 0).wait()
    pltpu.sync_copy(rows_v.at[0], agg_sh.at[idxd_v.at[0]], add=True)

    plsc.subcore_barrier()

    @pl.when(sid < 10)
    def _():
        for r4 in range(5):
            r0 = sid * 1000 + r4 * ZROWS
            pltpu.sync_copy(agg_sh.at[pl.ds(r0, ZROWS)],
                            aggout_hbm.at[cid, pl.ds(r0, ZROWS)])


@functools.partial(
    pl.kernel,
    out_type=jax.ShapeDtypeStruct((NC, N, D), _f32),
    mesh=_SC_MESH,
    scratch_types=[
        pltpu.VMEM((2, K), jnp.int32),
        pltpu.VMEM((2, K), jnp.int32),
        pltpu.VMEM((2, K, D), _f32),
        pltpu.VMEM((ZROWS, D), _f32),
        pltpu.VMEM_SHARED((N, D), _f32),
        pltpu.SemaphoreType.DMA,
        pltpu.SemaphoreType.DMA,
        pltpu.SemaphoreType.DMA,
        pltpu.SemaphoreType.DMA,
        pltpu.SemaphoreType.DMA,
        pltpu.SemaphoreType.DMA,
    ],
)
def _sc_message(src_hbm, dst_hbm, g_hbm, aggout_hbm,
                idxs_v, idxd_v, rows_v, zb_v, agg_sh,
                sAs0, sAs1, sAd0, sAd1, sB0, sB1):
    _msg_body(src_hbm, dst_hbm, g_hbm, aggout_hbm,
              idxs_v, idxd_v, rows_v, zb_v, agg_sh,
              sAs0, sAs1, sAd0, sAd1, sB0, sB1)


def _bil_body(i0_hbm, i1_hbm, hf_hbm, t_hbm, bb_hbm, out_hbm,
              idx0_v, idx1_v, r1_v, r2_v, sv_v, bb_v,
              sA00, sA01, sA10, sA11, sB00, sB01, sB10, sB11, sD0, sD1):
    cid = lax.axis_index("c")
    sid = lax.axis_index("s")
    w = sid * NC + cid
    base = w * EW
    sA0 = [sA00, sA01]
    sA1 = [sA10, sA11]
    sB0 = [sB00, sB01]
    sB1 = [sB10, sB11]
    sD = [sD0, sD1]

    pltpu.sync_copy(bb_hbm, bb_v)
    bbvec = bb_v[...]
    lane = lax.iota(jnp.int32, 16)
    perms = [jnp.bitwise_xor(lane, 1 << p) for p in range(4)]

    def _allsum(v):
        for p in perms:
            v = v + v.at[p].get(mode="promise_in_bounds")
        return v

    def cpA0(c, s):
        return pltpu.make_async_copy(
            i0_hbm.at[pl.ds(base + c * K, K)], idx0_v.at[s], sA0[s])

    def cpA1(c, s):
        return pltpu.make_async_copy(
            i1_hbm.at[pl.ds(base + c * K, K)], idx1_v.at[s], sA1[s])

    def cpB0(c, s):
        return pltpu.make_async_copy(
            hf_hbm.at[idx0_v.at[s]], r1_v.at[s], sB0[s])

    def cpB1(c, s):
        return pltpu.make_async_copy(
            t_hbm.at[idx1_v.at[s]], r2_v.at[s], sB1[s])

    def cpD(c, s):
        return pltpu.make_async_copy(
            sv_v.at[s], out_hbm.at[pl.ds(base + c * K, K)], sD[s])

    def compute(s):
        def group(gidx, _):
            acc = jnp.zeros((16,), _f32)
            for e in range(16):
                edge = gidx * 16 + e
                v = (r1_v[s, edge, pl.ds(0, 16)]
                     * r2_v[s, edge, pl.ds(0, 16)])
                for j in range(1, D // 16):
                    v = v + (r1_v[s, edge, pl.ds(j * 16, 16)]
                             * r2_v[s, edge, pl.ds(j * 16, 16)])
                acc = jnp.where(lane == e, _allsum(v), acc)
            sv_v[s, pl.ds(gidx * 16, 16)] = acc + bbvec
            return _

        lax.fori_loop(0, K // 16, group, None)

    cpA0(0, 0).start(); cpA1(0, 0).start()
    cpA0(1, 1).start(); cpA1(1, 1).start()
    cpA0(0, 0).wait(); cpA1(0, 0).wait()
    cpB0(0, 0).start(); cpB1(0, 0).start()

    def pair(i, _):
        c = i * 2
        for s in (0, 1):
            cc = c + s
            ns = 1 - s
            cpB0(cc, s).wait()
            cpB1(cc, s).wait()
            @pl.when(cc + 1 < NCHUNK)
            def _():
                cpA0(cc + 1, ns).wait()
                cpA1(cc + 1, ns).wait()
                cpB0(cc + 1, ns).start()
                cpB1(cc + 1, ns).start()
            @pl.when(cc >= 2)
            def _():
                cpD(cc - 2, s).wait()
            compute(s)
            cpD(cc, s).start()
            @pl.when(cc + 2 < NCHUNK)
            def _():
                cpA0(cc + 2, s).start()
                cpA1(cc + 2, s).start()
        return _

    lax.fori_loop(0, NCHUNK // 2, pair, None)
    cc = NCHUNK - 1
    cpB0(cc, 0).wait()
    cpB1(cc, 0).wait()
    cpD(cc - 2, 0).wait()
    compute(0)
    cpD(cc, 0).start()
    cpD(cc - 1, 1).wait()
    cpD(cc, 0).wait()


@functools.partial(
    pl.kernel,
    out_type=jax.ShapeDtypeStruct((E,), _f32),
    mesh=_SC_MESH,
    scratch_types=[
        pltpu.VMEM((2, K), jnp.int32),
        pltpu.VMEM((2, K), jnp.int32),
        pltpu.VMEM((2, K, D), _f32),
        pltpu.VMEM((2, K, D), _f32),
        pltpu.VMEM((2, K), _f32),
        pltpu.VMEM((16,), _f32),
        pltpu.SemaphoreType.DMA,
        pltpu.SemaphoreType.DMA,
        pltpu.SemaphoreType.DMA,
        pltpu.SemaphoreType.DMA,
        pltpu.SemaphoreType.DMA,
        pltpu.SemaphoreType.DMA,
        pltpu.SemaphoreType.DMA,
        pltpu.SemaphoreType.DMA,
        pltpu.SemaphoreType.DMA,
        pltpu.SemaphoreType.DMA,
    ],
)
def _sc_bilinear(i0_hbm, i1_hbm, hf_hbm, t_hbm, bb_hbm, out_hbm,
                 idx0_v, idx1_v, r1_v, r2_v, sv_v, bb_v,
                 sA00, sA01, sA10, sA11, sB00, sB01, sB10, sB11, sD0, sD1):
    _bil_body(i0_hbm, i1_hbm, hf_hbm, t_hbm, bb_hbm, out_hbm,
              idx0_v, idx1_v, r1_v, r2_v, sv_v, bb_v,
              sA00, sA01, sA10, sA11, sB00, sB01, sB10, sB11, sD0, sD1)


def kernel(x_input, edge_index_input, pos_edge_index_input,
           Wl, bl, Wg, bg, Wb, bb):
    src = pos_edge_index_input[0]
    dst = pos_edge_index_input[1]
    i0 = edge_index_input[0]
    i1 = edge_index_input[1]
    bl2 = bl.reshape(1, D)
    bg2 = bg.reshape(1, D)
    Wb0 = Wb[0]
    bb16 = jnp.broadcast_to(bb.astype(_f32), (16,))

    h2 = _tc_h2(x_input, Wl, bl2, Wg)
    degp = _sc_degree(dst)
    d0 = degp[0:N].reshape(N, 1)
    d1 = degp[NPAD:NPAD + N].reshape(N, 1)
    g = _tc_prescale(h2, d0, d1)
    aggp = _sc_message(src, dst, g)
    hf, t = _tc_finalize(aggp[0], aggp[1], g, d0, d1, bg2, Wb0)
    scores = _sc_bilinear(i0, i1, hf, t, bb16)
    return scores


# speedup vs baseline: 18.5435x; 1.2739x over previous
import functools

import jax
import jax.numpy as jnp
from jax import lax
from jax.experimental import pallas as pl
from jax.experimental.pallas import tpu as pltpu
from jax.experimental.pallas import tpu_sc as plsc

N = 10000
E = 320000
D = 128

NC = 2
NS = 16
NW = NC * NS
K = 128
CB = 78
NXTRA = E // K - NW * CB
IDXL = (CB + 1) * K

_f32 = jnp.float32


def _k1_body(x_ref, wl_ref, bl_ref, wg_ref, h2_ref):
    h = jnp.dot(x_ref[...], wl_ref[...], preferred_element_type=_f32)
    h = h + bl_ref[...]
    h2_ref[...] = jnp.dot(h, wg_ref[...], preferred_element_type=_f32)


def _tc_h2(x, Wl, bl2, Wg):
    blk = 1000
    return pl.pallas_call(
        _k1_body,
        grid=(N // blk,),
        in_specs=[
            pl.BlockSpec((blk, D), lambda i: (i, 0)),
            pl.BlockSpec((D, D), lambda i: (0, 0)),
            pl.BlockSpec((1, D), lambda i: (0, 0)),
            pl.BlockSpec((D, D), lambda i: (0, 0)),
        ],
        out_specs=pl.BlockSpec((blk, D), lambda i: (i, 0)),
        out_shape=jax.ShapeDtypeStruct((N, D), _f32),
    )(x, Wl, bl2, Wg)


def _k2_body(h2_ref, d0_ref, d1_ref, g_ref):
    deg = d0_ref[...] + d1_ref[...] + 1.0
    dinv = lax.rsqrt(deg)
    g_ref[...] = h2_ref[...] * dinv


def _tc_prescale(h2, d0, d1):
    blk = 1000
    return pl.pallas_call(
        _k2_body,
        grid=(N // blk,),
        in_specs=[
            pl.BlockSpec((blk, D), lambda i: (i, 0)),
            pl.BlockSpec((blk, 1), lambda i: (i, 0)),
            pl.BlockSpec((blk, 1), lambda i: (i, 0)),
        ],
        out_specs=pl.BlockSpec((blk, D), lambda i: (i, 0)),
        out_shape=jax.ShapeDtypeStruct((N, D), _f32),
    )(h2, d0, d1)


def _k4_body(a0_ref, a1_ref, g_ref, d0_ref, d1_ref, bg_ref, wb_ref,
             hf_ref, t_ref):
    deg = d0_ref[...] + d1_ref[...] + 1.0
    dinv = lax.rsqrt(deg)
    hf = dinv * (a0_ref[...] + a1_ref[...] + g_ref[...]) + bg_ref[...]
    hf_ref[...] = hf
    t_ref[...] = lax.dot_general(hf, wb_ref[...],
                                 (((1,), (1,)), ((), ())),
                                 preferred_element_type=_f32)


def _tc_finalize(a0, a1, g, d0, d1, bg2, Wb0):
    blk = 1000
    return pl.pallas_call(
        _k4_body,
        grid=(N // blk,),
        in_specs=[
            pl.BlockSpec((blk, D), lambda i: (i, 0)),
            pl.BlockSpec((blk, D), lambda i: (i, 0)),
            pl.BlockSpec((blk, D), lambda i: (i, 0)),
            pl.BlockSpec((blk, 1), lambda i: (i, 0)),
            pl.BlockSpec((blk, 1), lambda i: (i, 0)),
            pl.BlockSpec((1, D), lambda i: (0, 0)),
            pl.BlockSpec((D, D), lambda i: (0, 0)),
        ],
        out_specs=[
            pl.BlockSpec((blk, D), lambda i: (i, 0)),
            pl.BlockSpec((blk, D), lambda i: (i, 0)),
        ],
        out_shape=[
            jax.ShapeDtypeStruct((N, D), _f32),
            jax.ShapeDtypeStruct((N, D), _f32),
        ],
    )(a0, a1, g, d0, d1, bg2, Wb0)


_SC_MESH = plsc.VectorSubcoreMesh(core_axis_name="c", subcore_axis_name="s")

NPAD = 10240


def _worker(cid, sid):
    w = sid * NC + cid
    bc = w * CB + jnp.minimum(w, NXTRA)
    nch = jnp.where(w < NXTRA, CB + 1, CB)
    return w, bc * K, nch


def _deg_body(dst_hbm, degout_hbm, idx_v, ones_v, zb_v, deg_sh,
              sA0, sA1, sC0, sC1):
    cid = lax.axis_index("c")
    sid = lax.axis_index("s")
    w, base, nch = _worker(cid, sid)
    sA = [sA0, sA1]
    sC = [sC0, sC1]

    zeros16 = jnp.zeros((16,), _f32)
    ones16 = jnp.ones((16,), _f32)
    for j in range(1024 // 16):
        zb_v[pl.ds(j * 16, 16)] = zeros16
    for j in range(K // 16):
        ones_v[pl.ds(j * 16, 16)] = ones16

    @pl.when(sid < 10)
    def _():
        pltpu.sync_copy(zb_v, deg_sh.at[pl.ds(sid * 1024, 1024)])

    plsc.subcore_barrier()

    def cpA(c, s):
        return pltpu.make_async_copy(
            dst_hbm.at[pl.ds(base + c * K, K)], idx_v.at[s], sA[s])

    def cpC(s):
        return pltpu.make_async_copy(ones_v, deg_sh.at[idx_v.at[s]], sC[s])

    cpA(0, 0).start()
    cpA(1, 1).start()

    def pair(i, _):
        c = i * 2
        for s in (0, 1):
            cc = c + s
            cpA(cc, s).wait()
            cpC(s).start(add=True)
            @pl.when(cc + 2 < nch)
            def _():
                cpC(s).wait()
                cpA(cc + 2, s).start()
        return _

    lax.fori_loop(0, CB // 2, pair, None)

    @pl.when(nch > CB)
    def _():
        cpA(CB, 0).wait()
        cpC(0).start(add=True)
        cpC(0).wait()

    @pl.when(nch <= CB)
    def _():
        cpC(0).wait()
    cpC(1).wait()

    plsc.subcore_barrier()

    @pl.when(sid < 10)
    def _():
        pltpu.sync_copy(deg_sh.at[pl.ds(sid * 1024, 1024)],
                        degout_hbm.at[pl.ds(cid * NPAD + sid * 1024, 1024)])


@functools.partial(
    pl.kernel,
    out_type=jax.ShapeDtypeStruct((NC * NPAD,), _f32),
    mesh=_SC_MESH,
    scratch_types=[
        pltpu.VMEM((2, K), jnp.int32),
        pltpu.VMEM((K,), _f32),
        pltpu.VMEM((1024,), _f32),
        pltpu.VMEM_SHARED((NPAD,), _f32),
        pltpu.SemaphoreType.DMA,
        pltpu.SemaphoreType.DMA,
        pltpu.SemaphoreType.DMA,
        pltpu.SemaphoreType.DMA,
    ],
)
def _sc_degree(dst_hbm, degout_hbm, idx_v, ones_v, zb_v, deg_sh,
               sA0, sA1, sC0, sC1):
    _deg_body(dst_hbm, degout_hbm, idx_v, ones_v, zb_v, deg_sh,
              sA0, sA1, sC0, sC1)


ZROWS = 200


def _msg_body(src_hbm, dst_hbm, g_hbm, aggout_hbm,
              srcall_v, idxd_v, rows_v, agg_sh,
              sP, sAd0, sAd1, sB0, sB1):
    cid = lax.axis_index("c")
    sid = lax.axis_index("s")
    w, base, nch = _worker(cid, sid)
    sAd = [sAd0, sAd1]
    sB = [sB0, sB1]

    pltpu.async_copy(src_hbm.at[pl.ds(base, CB * K)],
                     srcall_v.at[pl.ds(0, CB * K)], sP).wait()
    @pl.when(nch > CB)
    def _():
        pltpu.async_copy(src_hbm.at[pl.ds(base + CB * K, K)],
                         srcall_v.at[pl.ds(CB * K, K)], sP).wait()

    zeros16 = jnp.zeros((16,), _f32)

    def zrow(r, _):
        for j in range(D // 16):
            rows_v[0, r, pl.ds(j * 16, 16)] = zeros16
        return _

    lax.fori_loop(0, K, zrow, None)

    @pl.when(sid < 10)
    def _():
        for r8 in range(8):
            pltpu.sync_copy(
                rows_v.at[0, pl.ds(0, 125)],
                agg_sh.at[pl.ds(sid * 1000 + r8 * 125, 125)])
    plsc.subcore_barrier()

    def cpAd(c, s):
        return pltpu.make_async_copy(
            dst_hbm.at[pl.ds(base + c * K, K)], idxd_v.at[s], sAd[s])

    def cpB(c, s):
        return pltpu.make_async_copy(
            g_hbm.at[srcall_v.at[pl.ds(c * K, K)]], rows_v.at[s], sB[s])

    cpAd(0, 0).start()
    cpAd(1, 1).start()
    cpB(0, 0).start()

    def pair(i, _):
        c = i * 2
        for s in (0, 1):
            cc = c + s
            ns = 1 - s
            cpB(cc, s).wait()
            @pl.when(cc + 1 < nch)
            def _():
                cpB(cc + 1, ns).start()
            cpAd(cc, s).wait()
            pltpu.sync_copy(rows_v.at[s], agg_sh.at[idxd_v.at[s]], add=True)
            @pl.when(cc + 2 < nch)
            def _():
                cpAd(cc + 2, s).start()
        return _

    lax.fori_loop(0, CB // 2, pair, None)

    @pl.when(nch > CB)
    def _():
        cpB(CB, 0).wait()
        cpAd(CB, 0).wait()
        pltpu.sync_copy(rows_v.at[0], agg_sh.at[idxd_v.at[0]], add=True)

    plsc.subcore_barrier()

    @pl.when(sid < 10)
    def _():
        for r4 in range(5):
            r0 = sid * 1000 + r4 * ZROWS
            pltpu.sync_copy(agg_sh.at[pl.ds(r0, ZROWS)],
                            aggout_hbm.at[cid, pl.ds(r0, ZROWS)])


@functools.partial(
    pl.kernel,
    out_type=jax.ShapeDtypeStruct((NC, N, D), _f32),
    mesh=_SC_MESH,
    scratch_types=[
        pltpu.VMEM((IDXL,), jnp.int32),
        pltpu.VMEM((2, K), jnp.int32),
        pltpu.VMEM((2, K, D), _f32),
        pltpu.VMEM_SHARED((N, D), _f32),
        pltpu.SemaphoreType.DMA,
        pltpu.SemaphoreType.DMA,
        pltpu.SemaphoreType.DMA,
        pltpu.SemaphoreType.DMA,
        pltpu.SemaphoreType.DMA,
    ],
)
def _sc_message(src_hbm, dst_hbm, g_hbm, aggout_hbm,
                srcall_v, idxd_v, rows_v, agg_sh,
                sP, sAd0, sAd1, sB0, sB1):
    _msg_body(src_hbm, dst_hbm, g_hbm, aggout_hbm,
              srcall_v, idxd_v, rows_v, agg_sh,
              sP, sAd0, sAd1, sB0, sB1)


def _bil_body(i0_hbm, i1_hbm, hf_hbm, t_hbm, bb_hbm, out_hbm,
              idx0_v, idx1_v, r1_v, r2_v, sv_v, bb_v,
              sP0, sP1, sB00, sB01, sB10, sB11, sD0, sD1):
    cid = lax.axis_index("c")
    sid = lax.axis_index("s")
    w, base, nch = _worker(cid, sid)
    sB0 = [sB00, sB01]
    sB1 = [sB10, sB11]
    sD = [sD0, sD1]

    d0 = pltpu.async_copy(i0_hbm.at[pl.ds(base, CB * K)],
                          idx0_v.at[pl.ds(0, CB * K)], sP0)
    d1 = pltpu.async_copy(i1_hbm.at[pl.ds(base, CB * K)],
                          idx1_v.at[pl.ds(0, CB * K)], sP1)
    d0.wait()
    d1.wait()
    @pl.when(nch > CB)
    def _():
        pltpu.async_copy(i0_hbm.at[pl.ds(base + CB * K, K)],
                         idx0_v.at[pl.ds(CB * K, K)], sP0).wait()
        pltpu.async_copy(i1_hbm.at[pl.ds(base + CB * K, K)],
                         idx1_v.at[pl.ds(CB * K, K)], sP1).wait()

    pltpu.sync_copy(bb_hbm, bb_v)
    bbvec = bb_v[...]
    lane = lax.iota(jnp.int32, 16)
    perms = [jnp.bitwise_xor(lane, 1 << p) for p in range(4)]

    def _allsum(v):
        for p in perms:
            v = v + v.at[p].get(mode="promise_in_bounds")
        return v

    def cpB0(c, s):
        return pltpu.make_async_copy(
            hf_hbm.at[idx0_v.at[pl.ds(c * K, K)]], r1_v.at[s], sB0[s])

    def cpB1(c, s):
        return pltpu.make_async_copy(
            t_hbm.at[idx1_v.at[pl.ds(c * K, K)]], r2_v.at[s], sB1[s])

    def cpD(c, s):
        return pltpu.make_async_copy(
            sv_v.at[s], out_hbm.at[pl.ds(base + c * K, K)], sD[s])

    def compute(s):
        def group(gidx, _):
            acc = jnp.zeros((16,), _f32)
            for e in range(16):
                edge = gidx * 16 + e
                v = (r1_v[s, edge, pl.ds(0, 16)]
                     * r2_v[s, edge, pl.ds(0, 16)])
                for j in range(1, D // 16):
                    v = v + (r1_v[s, edge, pl.ds(j * 16, 16)]
                             * r2_v[s, edge, pl.ds(j * 16, 16)])
                acc = jnp.where(lane == e, _allsum(v), acc)
            sv_v[s, pl.ds(gidx * 16, 16)] = acc + bbvec
            return _

        lax.fori_loop(0, K // 16, group, None)

    cpB0(0, 0).start()
    cpB1(0, 0).start()

    def pair(i, _):
        c = i * 2
        for s in (0, 1):
            cc = c + s
            ns = 1 - s
            cpB0(cc, s).wait()
            cpB1(cc, s).wait()
            @pl.when(cc + 1 < nch)
            def _():
                cpB0(cc + 1, ns).start()
                cpB1(cc + 1, ns).start()
            @pl.when(cc >= 2)
            def _():
                cpD(cc - 2, s).wait()
            compute(s)
            cpD(cc, s).start()
        return _

    lax.fori_loop(0, CB // 2, pair, None)

    @pl.when(nch > CB)
    def _():
        cpD(CB - 2, 0).wait()
        cpB0(CB, 0).wait()
        cpB1(CB, 0).wait()
        compute(0)
        cpD(CB, 0).start()
        cpD(CB, 0).wait()

    @pl.when(nch <= CB)
    def _():
        cpD(CB - 2, 0).wait()
    cpD(CB - 1, 1).wait()


@functools.partial(
    pl.kernel,
    out_type=jax.ShapeDtypeStruct((E,), _f32),
    mesh=_SC_MESH,
    scratch_types=[
        pltpu.VMEM((IDXL,), jnp.int32),
        pltpu.VMEM((IDXL,), jnp.int32),
        pltpu.VMEM((2, K, D), _f32),
        pltpu.VMEM((2, K, D), _f32),
        pltpu.VMEM((2, K), _f32),
        pltpu.VMEM((16,), _f32),
        pltpu.SemaphoreType.DMA,
        pltpu.SemaphoreType.DMA,
        pltpu.SemaphoreType.DMA,
        pltpu.SemaphoreType.DMA,
        pltpu.SemaphoreType.DMA,
        pltpu.SemaphoreType.DMA,
        pltpu.SemaphoreType.DMA,
        pltpu.SemaphoreType.DMA,
    ],
)
def _sc_bilinear(i0_hbm, i1_hbm, hf_hbm, t_hbm, bb_hbm, out_hbm,
                 idx0_v, idx1_v, r1_v, r2_v, sv_v, bb_v,
                 sP0, sP1, sB00, sB01, sB10, sB11, sD0, sD1):
    _bil_body(i0_hbm, i1_hbm, hf_hbm, t_hbm, bb_hbm, out_hbm,
              idx0_v, idx1_v, r1_v, r2_v, sv_v, bb_v,
              sP0, sP1, sB00, sB01, sB10, sB11, sD0, sD1)


def kernel(x_input, edge_index_input, pos_edge_index_input,
           Wl, bl, Wg, bg, Wb, bb):
    src = pos_edge_index_input[0]
    dst = pos_edge_index_input[1]
    i0 = edge_index_input[0]
    i1 = edge_index_input[1]
    bl2 = bl.reshape(1, D)
    bg2 = bg.reshape(1, D)
    Wb0 = Wb[0]
    bb16 = jnp.broadcast_to(bb.astype(_f32), (16,))

    h2 = _tc_h2(x_input, Wl, bl2, Wg)
    degp = _sc_degree(dst)
    d0 = degp[0:N].reshape(N, 1)
    d1 = degp[NPAD:NPAD + N].reshape(N, 1)
    g = _tc_prescale(h2, d0, d1)
    aggp = _sc_message(src, dst, g)
    hf, t = _tc_finalize(aggp[0], aggp[1], g, d0, d1, bg2, Wb0)
    scores = _sc_bilinear(i0, i1, hf, t, bb16)
    return scores


# speedup vs baseline: 18.5688x; 1.0014x over previous
import functools

import jax
import jax.numpy as jnp
from jax import lax
from jax.experimental import pallas as pl
from jax.experimental.pallas import tpu as pltpu
from jax.experimental.pallas import tpu_sc as plsc

N = 10000
E = 320000
D = 128

NC = 2
NS = 16
NW = NC * NS
K = 128
CB = 78
NXTRA = E // K - NW * CB
IDXL = (CB + 1) * K

_f32 = jnp.float32


def _k1_body(x_ref, wl_ref, bl_ref, wg_ref, h2_ref):
    h = jnp.dot(x_ref[...], wl_ref[...], preferred_element_type=_f32)
    h = h + bl_ref[...]
    h2_ref[...] = jnp.dot(h, wg_ref[...], preferred_element_type=_f32)


def _tc_h2(x, Wl, bl2, Wg):
    blk = 1000
    return pl.pallas_call(
        _k1_body,
        grid=(N // blk,),
        in_specs=[
            pl.BlockSpec((blk, D), lambda i: (i, 0)),
            pl.BlockSpec((D, D), lambda i: (0, 0)),
            pl.BlockSpec((1, D), lambda i: (0, 0)),
            pl.BlockSpec((D, D), lambda i: (0, 0)),
        ],
        out_specs=pl.BlockSpec((blk, D), lambda i: (i, 0)),
        out_shape=jax.ShapeDtypeStruct((N, D), _f32),
    )(x, Wl, bl2, Wg)


def _k2_body(h2_ref, d0_ref, d1_ref, g_ref):
    deg = d0_ref[...] + d1_ref[...] + 1.0
    dinv = lax.rsqrt(deg)
    g_ref[...] = h2_ref[...] * dinv


def _tc_prescale(h2, d0, d1):
    blk = 1000
    return pl.pallas_call(
        _k2_body,
        grid=(N // blk,),
        in_specs=[
            pl.BlockSpec((blk, D), lambda i: (i, 0)),
            pl.BlockSpec((blk, 1), lambda i: (i, 0)),
            pl.BlockSpec((blk, 1), lambda i: (i, 0)),
        ],
        out_specs=pl.BlockSpec((blk, D), lambda i: (i, 0)),
        out_shape=jax.ShapeDtypeStruct((N, D), _f32),
    )(h2, d0, d1)


def _k4_body(a0_ref, a1_ref, g_ref, d0_ref, d1_ref, bg_ref, wb_ref,
             hf_ref, t_ref):
    deg = d0_ref[...] + d1_ref[...] + 1.0
    dinv = lax.rsqrt(deg)
    hf = dinv * (a0_ref[...] + a1_ref[...] + g_ref[...]) + bg_ref[...]
    hf_ref[...] = hf
    t_ref[...] = lax.dot_general(hf, wb_ref[...],
                                 (((1,), (1,)), ((), ())),
                                 preferred_element_type=_f32)


def _tc_finalize(a0, a1, g, d0, d1, bg2, Wb0):
    blk = 1000
    return pl.pallas_call(
        _k4_body,
        grid=(N // blk,),
        in_specs=[
            pl.BlockSpec((blk, D), lambda i: (i, 0)),
            pl.BlockSpec((blk, D), lambda i: (i, 0)),
            pl.BlockSpec((blk, D), lambda i: (i, 0)),
            pl.BlockSpec((blk, 1), lambda i: (i, 0)),
            pl.BlockSpec((blk, 1), lambda i: (i, 0)),
            pl.BlockSpec((1, D), lambda i: (0, 0)),
            pl.BlockSpec((D, D), lambda i: (0, 0)),
        ],
        out_specs=[
            pl.BlockSpec((blk, D), lambda i: (i, 0)),
            pl.BlockSpec((blk, D), lambda i: (i, 0)),
        ],
        out_shape=[
            jax.ShapeDtypeStruct((N, D), _f32),
            jax.ShapeDtypeStruct((N, D), _f32),
        ],
    )(a0, a1, g, d0, d1, bg2, Wb0)


_SC_MESH = plsc.VectorSubcoreMesh(core_axis_name="c", subcore_axis_name="s")

NPAD = 10240


def _worker(cid, sid):
    w = sid * NC + cid
    bc = w * CB + jnp.minimum(w, NXTRA)
    nch = jnp.where(w < NXTRA, CB + 1, CB)
    return w, bc * K, nch


def _deg_body(dst_hbm, degout_hbm, idx_v, ones_v, zb_v, deg_sh,
              sA0, sA1, sC0, sC1):
    cid = lax.axis_index("c")
    sid = lax.axis_index("s")
    w, base, nch = _worker(cid, sid)
    sA = [sA0, sA1]
    sC = [sC0, sC1]

    zeros16 = jnp.zeros((16,), _f32)
    ones16 = jnp.ones((16,), _f32)
    for j in range(1024 // 16):
        zb_v[pl.ds(j * 16, 16)] = zeros16
    for j in range(K // 16):
        ones_v[pl.ds(j * 16, 16)] = ones16

    @pl.when(sid < 10)
    def _():
        pltpu.sync_copy(zb_v, deg_sh.at[pl.ds(sid * 1024, 1024)])

    plsc.subcore_barrier()

    def cpA(c, s):
        return pltpu.make_async_copy(
            dst_hbm.at[pl.ds(base + c * K, K)], idx_v.at[s], sA[s])

    def cpC(s):
        return pltpu.make_async_copy(ones_v, deg_sh.at[idx_v.at[s]], sC[s])

    cpA(0, 0).start()
    cpA(1, 1).start()

    def pair(i, _):
        c = i * 2
        for s in (0, 1):
            cc = c + s
            cpA(cc, s).wait()
            cpC(s).start(add=True)
            @pl.when(cc + 2 < nch)
            def _():
                cpC(s).wait()
                cpA(cc + 2, s).start()
        return _

    lax.fori_loop(0, CB // 2, pair, None)

    @pl.when(nch > CB)
    def _():
        cpA(CB, 0).wait()
        cpC(0).start(add=True)
        cpC(0).wait()

    @pl.when(nch <= CB)
    def _():
        cpC(0).wait()
    cpC(1).wait()

    plsc.subcore_barrier()

    @pl.when(sid < 10)
    def _():
        pltpu.sync_copy(deg_sh.at[pl.ds(sid * 1024, 1024)],
                        degout_hbm.at[pl.ds(cid * NPAD + sid * 1024, 1024)])


@functools.partial(
    pl.kernel,
    out_type=jax.ShapeDtypeStruct((NC * NPAD,), _f32),
    mesh=_SC_MESH,
    scratch_types=[
        pltpu.VMEM((2, K), jnp.int32),
        pltpu.VMEM((K,), _f32),
        pltpu.VMEM((1024,), _f32),
        pltpu.VMEM_SHARED((NPAD,), _f32),
        pltpu.SemaphoreType.DMA,
        pltpu.SemaphoreType.DMA,
        pltpu.SemaphoreType.DMA,
        pltpu.SemaphoreType.DMA,
    ],
)
def _sc_degree(dst_hbm, degout_hbm, idx_v, ones_v, zb_v, deg_sh,
               sA0, sA1, sC0, sC1):
    _deg_body(dst_hbm, degout_hbm, idx_v, ones_v, zb_v, deg_sh,
              sA0, sA1, sC0, sC1)


ZROWS = 200


def _msg_body(src_hbm, dst_hbm, g_hbm, aggout_hbm,
              idxs_v, idxd_v, rows_v, agg_sh,
              sAs0, sAs1, sAs2, sAs3, sAd0, sAd1, sAd2, sAd3,
              sB0, sB1, sC0, sC1):
    cid = lax.axis_index("c")
    sid = lax.axis_index("s")
    w, base, nch = _worker(cid, sid)
    sAs = [sAs0, sAs1, sAs2, sAs3]
    sAd = [sAd0, sAd1, sAd2, sAd3]
    sB = [sB0, sB1]
    sC = [sC0, sC1]

    zeros16 = jnp.zeros((16,), _f32)

    def zrow(r, _):
        for j in range(D // 16):
            rows_v[0, r, pl.ds(j * 16, 16)] = zeros16
        return _

    lax.fori_loop(0, K, zrow, None)

    @pl.when(sid < 10)
    def _():
        for r8 in range(8):
            pltpu.sync_copy(
                rows_v.at[0, pl.ds(0, 125)],
                agg_sh.at[pl.ds(sid * 1000 + r8 * 125, 125)])
    plsc.subcore_barrier()

    def cpAs(c, q):
        off = jnp.minimum(base + c * K, E - K)
        return pltpu.make_async_copy(
            src_hbm.at[pl.ds(off, K)], idxs_v.at[q], sAs[q])

    def cpAd(c, q):
        off = jnp.minimum(base + c * K, E - K)
        return pltpu.make_async_copy(
            dst_hbm.at[pl.ds(off, K)], idxd_v.at[q], sAd[q])

    def cpB(q, s):
        return pltpu.make_async_copy(
            g_hbm.at[idxs_v.at[q]], rows_v.at[s], sB[s])

    def cpC(s, q):
        return pltpu.make_async_copy(
            rows_v.at[s], agg_sh.at[idxd_v.at[q]], sC[s])

    for q0 in (0, 1, 2):
        cpAs(q0, q0).start()
        cpAd(q0, q0).start()
    cpAs(0, 0).wait()
    cpB(0, 0).start()

    def bod(cc, s, q):
        ns = 1 - s
        cpB(q, s).wait()
        @pl.when(cc >= 1)
        def _():
            cpC(ns, (q + 3) % 4).wait()
        @pl.when(cc + 1 < nch)
        def _():
            cpAs((q + 1) % 4, (q + 1) % 4).wait()
            cpB((q + 1) % 4, ns).start()
        @pl.when(cc + 3 < nch)
        def _():
            cpAs(cc + 3, (q + 3) % 4).start()
            cpAd(cc + 3, (q + 3) % 4).start()
        cpAd(cc, q).wait()
        cpC(s, q).start(add=True)

    def quad(i, _):
        c = i * 4
        for j in range(4):
            bod(c + j, j % 2, j)
        return _

    lax.fori_loop(0, CB // 4, quad, None)
    bod(CB - 2, 0, 0)
    bod(CB - 1, 1, 1)

    @pl.when(nch > CB)
    def _():
        bod(CB, 0, 2)
        cpC(0, 2).wait()

    @pl.when(nch <= CB)
    def _():
        cpC(1, 1).wait()

    plsc.subcore_barrier()

    @pl.when(sid < 10)
    def _():
        for r4 in range(5):
            r0 = sid * 1000 + r4 * ZROWS
            pltpu.sync_copy(agg_sh.at[pl.ds(r0, ZROWS)],
                            aggout_hbm.at[cid, pl.ds(r0, ZROWS)])


@functools.partial(
    pl.kernel,
    out_type=jax.ShapeDtypeStruct((NC, N, D), _f32),
    mesh=_SC_MESH,
    scratch_types=[
        pltpu.VMEM((4, K), jnp.int32),
        pltpu.VMEM((4, K), jnp.int32),
        pltpu.VMEM((2, K, D), _f32),
        pltpu.VMEM_SHARED((N, D), _f32),
        pltpu.SemaphoreType.DMA,
        pltpu.SemaphoreType.DMA,
        pltpu.SemaphoreType.DMA,
        pltpu.SemaphoreType.DMA,
        pltpu.SemaphoreType.DMA,
        pltpu.SemaphoreType.DMA,
        pltpu.SemaphoreType.DMA,
        pltpu.SemaphoreType.DMA,
        pltpu.SemaphoreType.DMA,
        pltpu.SemaphoreType.DMA,
        pltpu.SemaphoreType.DMA,
        pltpu.SemaphoreType.DMA,
    ],
)
def _sc_message(src_hbm, dst_hbm, g_hbm, aggout_hbm,
                idxs_v, idxd_v, rows_v, agg_sh,
                sAs0, sAs1, sAs2, sAs3, sAd0, sAd1, sAd2, sAd3,
                sB0, sB1, sC0, sC1):
    _msg_body(src_hbm, dst_hbm, g_hbm, aggout_hbm,
              idxs_v, idxd_v, rows_v, agg_sh,
              sAs0, sAs1, sAs2, sAs3, sAd0, sAd1, sAd2, sAd3,
              sB0, sB1, sC0, sC1)


def _bil_body(i0_hbm, i1_hbm, hf_hbm, t_hbm, bb_hbm, out_hbm,
              idx0_v, idx1_v, r1_v, r2_v, sv_v, bb_v,
              sP0, sP1, sB00, sB01, sB10, sB11, sD0, sD1):
    cid = lax.axis_index("c")
    sid = lax.axis_index("s")
    w, base, nch = _worker(cid, sid)
    sB0 = [sB00, sB01]
    sB1 = [sB10, sB11]
    sD = [sD0, sD1]

    d0 = pltpu.async_copy(i0_hbm.at[pl.ds(base, CB * K)],
                          idx0_v.at[pl.ds(0, CB * K)], sP0)
    d1 = pltpu.async_copy(i1_hbm.at[pl.ds(base, CB * K)],
                          idx1_v.at[pl.ds(0, CB * K)], sP1)
    d0.wait()
    d1.wait()
    @pl.when(nch > CB)
    def _():
        pltpu.async_copy(i0_hbm.at[pl.ds(base + CB * K, K)],
                         idx0_v.at[pl.ds(CB * K, K)], sP0).wait()
        pltpu.async_copy(i1_hbm.at[pl.ds(base + CB * K, K)],
                         idx1_v.at[pl.ds(CB * K, K)], sP1).wait()

    pltpu.sync_copy(bb_hbm, bb_v)
    bbvec = bb_v[...]
    lane = lax.iota(jnp.int32, 16)
    perms = [jnp.bitwise_xor(lane, 1 << p) for p in range(4)]

    def _allsum(v):
        for p in perms:
            v = v + v.at[p].get(mode="promise_in_bounds")
        return v

    def cpB0(c, s):
        off = jnp.minimum(c, CB) * K
        return pltpu.make_async_copy(
            hf_hbm.at[idx0_v.at[pl.ds(off, K)]], r1_v.at[s], sB0[s])

    def cpB1(c, s):
        off = jnp.minimum(c, CB) * K
        return pltpu.make_async_copy(
            t_hbm.at[idx1_v.at[pl.ds(off, K)]], r2_v.at[s], sB1[s])

    def cpD(c, s):
        return pltpu.make_async_copy(
            sv_v.at[s], out_hbm.at[pl.ds(base + c * K, K)], sD[s])

    def compute(s):
        def group(gidx, _):
            acc = jnp.zeros((16,), _f32)
            for e in range(16):
                edge = gidx * 16 + e
                v = (r1_v[s, edge, pl.ds(0, 16)]
                     * r2_v[s, edge, pl.ds(0, 16)])
                for j in range(1, D // 16):
                    v = v + (r1_v[s, edge, pl.ds(j * 16, 16)]
                             * r2_v[s, edge, pl.ds(j * 16, 16)])
                acc = jnp.where(lane == e, _allsum(v), acc)
            sv_v[s, pl.ds(gidx * 16, 16)] = acc + bbvec
            return _

        lax.fori_loop(0, K // 16, group, None)

    cpB0(0, 0).start()
    cpB1(0, 0).start()

    def pair(i, _):
        c = i * 2
        for s in (0, 1):
            cc = c + s
            ns = 1 - s
            cpB0(cc, s).wait()
            cpB1(cc, s).wait()
            @pl.when(cc + 1 < nch)
            def _():
                cpB0(cc + 1, ns).start()
                cpB1(cc + 1, ns).start()
            @pl.when(cc >= 2)
            def _():
                cpD(cc - 2, s).wait()
            compute(s)
            cpD(cc, s).start()
        return _

    lax.fori_loop(0, CB // 2, pair, None)

    @pl.when(nch > CB)
    def _():
        cpD(CB - 2, 0).wait()
        cpB0(CB, 0).wait()
        cpB1(CB, 0).wait()
        compute(0)
        cpD(CB, 0).start()
        cpD(CB, 0).wait()

    @pl.when(nch <= CB)
    def _():
        cpD(CB - 2, 0).wait()
    cpD(CB - 1, 1).wait()


@functools.partial(
    pl.kernel,
    out_type=jax.ShapeDtypeStruct((E,), _f32),
    mesh=_SC_MESH,
    scratch_types=[
        pltpu.VMEM((IDXL,), jnp.int32),
        pltpu.VMEM((IDXL,), jnp.int32),
        pltpu.VMEM((2, K, D), _f32),
        pltpu.VMEM((2, K, D), _f32),
        pltpu.VMEM((2, K), _f32),
        pltpu.VMEM((16,), _f32),
        pltpu.SemaphoreType.DMA,
        pltpu.SemaphoreType.DMA,
        pltpu.SemaphoreType.DMA,
        pltpu.SemaphoreType.DMA,
        pltpu.SemaphoreType.DMA,
        pltpu.SemaphoreType.DMA,
        pltpu.SemaphoreType.DMA,
        pltpu.SemaphoreType.DMA,
    ],
)
def _sc_bilinear(i0_hbm, i1_hbm, hf_hbm, t_hbm, bb_hbm, out_hbm,
                 idx0_v, idx1_v, r1_v, r2_v, sv_v, bb_v,
                 sP0, sP1, sB00, sB01, sB10, sB11, sD0, sD1):
    _bil_body(i0_hbm, i1_hbm, hf_hbm, t_hbm, bb_hbm, out_hbm,
              idx0_v, idx1_v, r1_v, r2_v, sv_v, bb_v,
              sP0, sP1, sB00, sB01, sB10, sB11, sD0, sD1)


def kernel(x_input, edge_index_input, pos_edge_index_input,
           Wl, bl, Wg, bg, Wb, bb):
    src = pos_edge_index_input[0]
    dst = pos_edge_index_input[1]
    i0 = edge_index_input[0]
    i1 = edge_index_input[1]
    bl2 = bl.reshape(1, D)
    bg2 = bg.reshape(1, D)
    Wb0 = Wb[0]
    bb16 = jnp.broadcast_to(bb.astype(_f32), (16,))

    h2 = _tc_h2(x_input, Wl, bl2, Wg)
    degp = _sc_degree(dst)
    d0 = degp[0:N].reshape(N, 1)
    d1 = degp[NPAD:NPAD + N].reshape(N, 1)
    g = _tc_prescale(h2, d0, d1)
    aggp = _sc_message(src, dst, g)
    hf, t = _tc_finalize(aggp[0], aggp[1], g, d0, d1, bg2, Wb0)
    scores = _sc_bilinear(i0, i1, hf, t, bb16)
    return scores
